# Initial kernel scaffold; baseline (speedup 1.0000x reference)
import jax
import jax.numpy as jnp
from jax.experimental import pallas as pl


def kernel(in_features, edge_index, W_self1, W_neigh1, b1, W_self2, W_neigh2, b2):
    raise NotImplementedError("write your pallas kernel here")


# speedup vs baseline: 5.0152x; 5.0152x over previous
import functools

import jax
import jax.numpy as jnp
from jax import lax
from jax.experimental import pallas as pl
from jax.experimental.pallas import tpu as pltpu
from jax.experimental.pallas import tpu_sc as plsc

NC = 2
NS = 16
LANES = 16
CH = 128


def _edge_scatter_body(nchunk, rpw, perw, table, src, dst, zeros, out,
                       srcv, dstv, rows, acc, sem):
    c = lax.axis_index("c")
    s = lax.axis_index("s")
    pltpu.sync_copy(zeros, acc.at[pl.ds(s * rpw, rpw)])
    plsc.subcore_barrier()
    base = c * (nchunk // NC) + s * perw

    def step(i, carry):
        chunk = base + i
        pltpu.sync_copy(src.at[chunk], srcv)
        pltpu.sync_copy(dst.at[chunk], dstv)
        pltpu.async_copy(table.at[srcv], rows, sem).wait()
        pltpu.sync_copy(rows, acc.at[dstv], add=True)
        return carry

    lax.fori_loop(0, perw, step, 0)
    plsc.subcore_barrier()
    pltpu.sync_copy(acc.at[pl.ds(s * rpw, rpw)],
                    out.at[c, pl.ds(s * rpw, rpw)])


def _make_edge_scatter(nchunk, r, w):
    perw = nchunk // (NC * NS)
    rpw = r // NS
    mesh = plsc.VectorSubcoreMesh(core_axis_name="c", subcore_axis_name="s",
                                  num_cores=NC, num_subcores=NS)
    body = functools.partial(_edge_scatter_body, nchunk, rpw, perw)
    return pl.kernel(
        body,
        out_type=jax.ShapeDtypeStruct((NC, r, w), jnp.float32),
        mesh=mesh,
        scratch_types=[
            pltpu.VMEM((CH,), jnp.int32),
            pltpu.VMEM((CH,), jnp.int32),
            pltpu.VMEM((CH, w), jnp.float32),
            pltpu.VMEM_SHARED((r, w), jnp.float32),
            pltpu.SemaphoreType.DMA,
        ],
        compiler_params=pltpu.CompilerParams(use_tc_tiling_on_sc=False),
    )


def kernel(in_features, edge_index, W_self1, W_neigh1, b1,
           W_self2, W_neigh2, b2):
    x = in_features
    n, d = x.shape
    e = edge_index.shape[1]
    ncls = W_self2.shape[1]
    w1 = d + LANES
    w2 = -(-ncls // LANES) * LANES
    nchunk = -(-e // (CH * NC * NS)) * (NC * NS)
    epad = nchunk * CH
    r = -(-(n + 1) // (NS * 8)) * (NS * 8)
    blk = 1000
    g = n // blk

    src = jnp.concatenate(
        [edge_index[0], jnp.zeros((epad - e,), jnp.int32)]).reshape(nchunk, CH)
    dst = jnp.concatenate(
        [edge_index[1], jnp.full((epad - e,), n, jnp.int32)]).reshape(nchunk, CH)

    def body_a(x_ref, ws_ref, wn_ref, xs_ref, pe_ref):
        xb = x_ref[:]
        xs_ref[:] = jnp.dot(xb, ws_ref[:], preferred_element_type=jnp.float32)
        p = jnp.dot(xb, wn_ref[:], preferred_element_type=jnp.float32)
        pe_ref[:] = jnp.concatenate(
            [p, jnp.ones((blk, LANES), jnp.float32)], axis=1)

    xs, p1e = pl.pallas_call(
        body_a, grid=(g,),
        in_specs=[pl.BlockSpec((blk, d), lambda i: (i, 0)),
                  pl.BlockSpec((d, d), lambda i: (0, 0)),
                  pl.BlockSpec((d, d), lambda i: (0, 0))],
        out_specs=[pl.BlockSpec((blk, d), lambda i: (i, 0)),
                   pl.BlockSpec((blk, w1), lambda i: (i, 0))],
        out_shape=[jax.ShapeDtypeStruct((n, d), jnp.float32),
                   jax.ShapeDtypeStruct((n, w1), jnp.float32)],
    )(x, W_self1, W_neigh1)

    z1 = jnp.zeros((r // NS, w1), jnp.float32)
    s1 = _make_edge_scatter(nchunk, r, w1)(p1e, src, dst, z1)

    ws2p = jnp.pad(W_self2, ((0, 0), (0, w2 - ncls)))
    wn2p = jnp.pad(W_neigh2, ((0, 0), (0, w2 - ncls)))
    b1r = b1.reshape(1, d)

    def body_b(xs_ref, sa_ref, sb_ref, b1_ref, ws2_ref, wn2_ref,
               hs_ref, p2_ref, di_ref):
        sa = sa_ref[0]
        sb = sb_ref[0]
        deg = sa[:, d:d + 1] + sb[:, d:d + 1]
        dinv = 1.0 / jnp.maximum(deg, 1.0)
        agg = (sa[:, :d] + sb[:, :d]) * dinv
        h = jnp.maximum(xs_ref[:] + agg + b1_ref[:], 0.0)
        hs_ref[:] = jnp.dot(h, ws2_ref[:], preferred_element_type=jnp.float32)
        p2_ref[:] = jnp.dot(h, wn2_ref[:], preferred_element_type=jnp.float32)
        di_ref[:] = jnp.broadcast_to(dinv, (blk, w2))

    hs, p2, dinv = pl.pallas_call(
        body_b, grid=(g,),
        in_specs=[pl.BlockSpec((blk, d), lambda i: (i, 0)),
                  pl.BlockSpec((1, blk, w1), lambda i: (0, i, 0)),
                  pl.BlockSpec((1, blk, w1), lambda i: (1, i, 0)),
                  pl.BlockSpec((1, d), lambda i: (0, 0)),
                  pl.BlockSpec((d, w2), lambda i: (0, 0)),
                  pl.BlockSpec((d, w2), lambda i: (0, 0))],
        out_specs=[pl.BlockSpec((blk, w2), lambda i: (i, 0)),
                   pl.BlockSpec((blk, w2), lambda i: (i, 0)),
                   pl.BlockSpec((blk, w2), lambda i: (i, 0))],
        out_shape=[jax.ShapeDtypeStruct((n, w2), jnp.float32),
                   jax.ShapeDtypeStruct((n, w2), jnp.float32),
                   jax.ShapeDtypeStruct((n, w2), jnp.float32)],
    )(xs, s1, s1, b1r, ws2p, wn2p)

    z2 = jnp.zeros((r // NS, w2), jnp.float32)
    s2 = _make_edge_scatter(nchunk, r, w2)(p2, src, dst, z2)

    b2r = jnp.pad(b2, (0, w2 - ncls)).reshape(1, w2)

    def body_c(hs_ref, ta_ref, tb_ref, di_ref, b2_ref, o_ref):
        aggv = (ta_ref[0] + tb_ref[0]) * di_ref[:]
        res = hs_ref[:] + aggv + b2_ref[:]
        o_ref[:] = res[:, :ncls]

    out = pl.pallas_call(
        body_c, grid=(g,),
        in_specs=[pl.BlockSpec((blk, w2), lambda i: (i, 0)),
                  pl.BlockSpec((1, blk, w2), lambda i: (0, i, 0)),
                  pl.BlockSpec((1, blk, w2), lambda i: (1, i, 0)),
                  pl.BlockSpec((blk, w2), lambda i: (i, 0)),
                  pl.BlockSpec((1, w2), lambda i: (0, 0))],
        out_specs=pl.BlockSpec((blk, ncls), lambda i: (i, 0)),
        out_shape=jax.ShapeDtypeStruct((n, ncls), jnp.float32),
    )(hs, s2, s2, dinv, b2r)
    return out
